# baseline (speedup 1.0000x reference)
import functools

import jax
import jax.numpy as jnp
from jax import lax
from jax.experimental import pallas as pl
from jax.experimental.pallas import tpu as pltpu
from jax.experimental.pallas import tpu_sc as plsc

N = 10000
E = 320000
HEADS = 4
GAT_OUT = 256
N_GRAPHS = 64
BR = 1000

NP = 10240
EP = 327680
EPG = 331776
EB = 128


def _sc_gcn_msg(p_planes, src3, dst3, n_planes):
    n_chunks = src3.shape[0]
    nblk = src3.shape[1]
    rpt = NP // 16

    mesh = plsc.VectorSubcoreMesh(core_axis_name="c", subcore_axis_name="s")

    @functools.partial(
        pl.kernel,
        mesh=mesh,
        compiler_params=pltpu.CompilerParams(needs_layout_passes=False),
        out_type=jax.ShapeDtypeStruct((2, NP, 128), jnp.float32),
        scratch_types=[
            pltpu.VMEM((2, EB), jnp.int32),
            pltpu.VMEM((2, EB), jnp.int32),
            pltpu.VMEM((EB, 128), jnp.float32),
            pltpu.VMEM((EB, 128), jnp.float32),
            pltpu.VMEM_SHARED((NP, 128), jnp.float32),
            pltpu.SemaphoreType.DMA,
            pltpu.SemaphoreType.DMA,
        ],
    )
    def kmsg(p_hbm, src_hbm, dst_hbm, out_hbm, idxa, idxb, rowa, rowb,
             acc, sema, semb):
        c = lax.axis_index("c")
        s = lax.axis_index("s")
        chunk = s if n_chunks == 16 else c * 16 + s
        plane = c if n_planes == 2 else 0

        zero = jnp.zeros((16,), jnp.float32)

        def zbody(r, _):
            for j in range(8):
                rowa[r, pl.ds(j * 16, 16)] = zero
            return 0

        lax.fori_loop(0, EB, zbody, 0)
        rbase = s * rpt
        for t in range(rpt // EB):
            pltpu.sync_copy(rowa, acc.at[pl.ds(rbase + t * EB, EB)])
        plsc.subcore_barrier()

        idx = [idxa, idxb]
        row = [rowa, rowb]
        sem = [sema, semb]

        def load_and_fire(blk, b):
            pltpu.sync_copy(src_hbm.at[chunk].at[blk], idx[b].at[0])
            pltpu.sync_copy(dst_hbm.at[chunk].at[blk], idx[b].at[1])
            pltpu.async_copy(p_hbm.at[plane].at[idx[b].at[0]],
                             row[b], sem[b])

        def drain_and_scatter(b):
            pltpu.make_async_copy(p_hbm.at[plane].at[idx[b].at[0]],
                                  row[b], sem[b]).wait()
            pltpu.sync_copy(row[b], acc.at[idx[b].at[1]], add=True)

        load_and_fire(0, 0)

        def body(kk, _):
            blk2 = kk * 2
            load_and_fire(blk2 + 1, 1)
            drain_and_scatter(0)
            nxt = jnp.minimum(blk2 + 2, nblk - 1)
            load_and_fire(nxt, 0)
            drain_and_scatter(1)
            return 0

        lax.fori_loop(0, nblk // 2, body, 0)
        pltpu.make_async_copy(p_hbm.at[plane].at[idxa.at[0]],
                              rowa, sema).wait()
        plsc.subcore_barrier()
        pltpu.sync_copy(acc.at[pl.ds(rbase, rpt)],
                        out_hbm.at[c].at[pl.ds(rbase, rpt)])

    return kmsg(p_planes, src3, dst3)


def _mm(x, w):
    n, k = x.shape
    m = w.shape[1]
    br = 1024 if n % 1024 == 0 else BR

    def body(x_ref, w_ref, o_ref):
        o_ref[...] = jnp.dot(x_ref[...], w_ref[...],
                             preferred_element_type=jnp.float32)

    return pl.pallas_call(
        body,
        grid=(n // br,),
        in_specs=[
            pl.BlockSpec((br, k), lambda i: (i, 0)),
            pl.BlockSpec((k, m), lambda i: (0, 0)),
        ],
        out_specs=pl.BlockSpec((br, m), lambda i: (i, 0)),
        out_shape=jax.ShapeDtypeStruct((n, m), jnp.float32),
    )(x, w)


def _col_stats(y):
    n, m = y.shape

    def body(y_ref, s_ref):
        @pl.when(pl.program_id(0) == 0)
        def _():
            s_ref[...] = jnp.zeros_like(s_ref)

        yb = y_ref[...]
        s_ref[0:1, :] += jnp.sum(yb, 0, keepdims=True)
        s_ref[1:2, :] += jnp.sum(yb * yb, 0, keepdims=True)

    return pl.pallas_call(
        body,
        grid=(n // BR,),
        in_specs=[pl.BlockSpec((BR, m), lambda i: (i, 0))],
        out_specs=pl.BlockSpec((8, m), lambda i: (0, 0)),
        out_shape=jax.ShapeDtypeStruct((8, m), jnp.float32),
    )(y)


def _bn_affine(stats, g, be):
    s1 = stats[0]
    s2 = stats[1]
    mean = s1 / N
    var = s2 / N - mean * mean
    a = g * lax.rsqrt(var + 1e-5)
    c = be - mean * a
    return a, c


def _sc_gat_coef(a_s2, a_d2, src16, dst16):
    nblk = src16.shape[1]
    seg = NP // 16

    mesh = plsc.VectorSubcoreMesh(core_axis_name="c", subcore_axis_name="s")

    @functools.partial(
        pl.kernel,
        mesh=mesh,
        compiler_params=pltpu.CompilerParams(needs_layout_passes=False),
        out_type=(
            jax.ShapeDtypeStruct((16, nblk, HEADS, 128), jnp.float32),
            jax.ShapeDtypeStruct((HEADS, NP), jnp.float32),
        ),
        scratch_types=[
            pltpu.VMEM((NP,), jnp.float32),
            pltpu.VMEM((NP,), jnp.float32),
            pltpu.VMEM((NP,), jnp.float32),
            pltpu.VMEM((NP,), jnp.float32),
            pltpu.VMEM((NP,), jnp.float32),
            pltpu.VMEM((NP,), jnp.float32),
            pltpu.VMEM((NP,), jnp.float32),
            pltpu.VMEM((NP,), jnp.float32),
            pltpu.VMEM((128,), jnp.int32),
            pltpu.VMEM((128,), jnp.int32),
            pltpu.VMEM((2, 128), jnp.float32),
            pltpu.VMEM((seg,), jnp.float32),
            pltpu.VMEM((seg,), jnp.float32),
            pltpu.VMEM((seg,), jnp.float32),
            pltpu.VMEM_SHARED((16, 2, NP), jnp.float32),
            pltpu.VMEM_SHARED((2, NP), jnp.float32),
        ],
    )
    def kcoef(as_hbm, ad_hbm, src_hbm, dst_hbm, coef_hbm, dd_hbm,
          asv0, asv1, adv0, adv1, denl0, denl1, ddv0, ddv1,
          srcv, dstv, cfr, sum0, sum1, tmpv, den_sh, dd_sh):
        c = lax.axis_index("c")
        s = lax.axis_index("s")
        asv = [asv0, asv1]
        adv = [adv0, adv1]
        denl = [denl0, denl1]
        ddv = [ddv0, ddv1]
        sumv = [sum0, sum1]
        for h in range(2):
            pltpu.sync_copy(as_hbm.at[c].at[h], asv[h])
            pltpu.sync_copy(ad_hbm.at[c].at[h], adv[h])

        zero = jnp.zeros((16,), jnp.float32)

        def zb(i, _):
            denl0[pl.ds(i * 16, 16)] = zero
            denl1[pl.ds(i * 16, 16)] = zero
            return 0

        lax.fori_loop(0, NP // 16, zb, 0)

        def edge_ex(h, j):
            s16 = srcv[pl.ds(j * 16, 16)]
            d16 = dstv[pl.ds(j * 16, 16)]
            a16 = plsc.load_gather(asv[h], [s16])
            b16 = plsc.load_gather(adv[h], [d16])
            e16 = a16 + b16
            e16 = jnp.maximum(e16, 0.2 * e16)
            return jnp.exp(e16), d16

        def pass1(kk, _):
            pltpu.sync_copy(src_hbm.at[s].at[kk], srcv)
            pltpu.sync_copy(dst_hbm.at[s].at[kk], dstv)
            for h in range(2):
                for j in range(8):
                    ex16, d16 = edge_ex(h, j)
                    plsc.addupdate_scatter(denl[h], [d16], ex16)
            return 0

        lax.fori_loop(0, nblk, pass1, 0)
        for h in range(2):
            pltpu.sync_copy(denl[h], den_sh.at[s].at[h])
        plsc.subcore_barrier()

        cb = s * seg
        for h in range(2):
            sv = sumv[h]
            pltpu.sync_copy(den_sh.at[0].at[h].at[pl.ds(cb, seg)], tmpv)

            def cpy(i, _):
                sv[pl.ds(i * 16, 16)] = tmpv[pl.ds(i * 16, 16)]
                return 0

            lax.fori_loop(0, seg // 16, cpy, 0)

            def red_t(t, _):
                pltpu.sync_copy(den_sh.at[t].at[h].at[pl.ds(cb, seg)], tmpv)

                def add_i(i, _):
                    sl = pl.ds(i * 16, 16)
                    sv[sl] += tmpv[sl]
                    return 0

                lax.fori_loop(0, seg // 16, add_i, 0)
                return 0

            lax.fori_loop(1, 16, red_t, 0)

            def inv_i(i, _):
                sl = pl.ds(i * 16, 16)
                sv[sl] = 1.0 / (sv[sl] + 1e-16)
                return 0

            lax.fori_loop(0, seg // 16, inv_i, 0)
            pltpu.sync_copy(sv, dd_sh.at[h].at[pl.ds(cb, seg)])
            pltpu.sync_copy(sv, dd_hbm.at[2 * c + h].at[pl.ds(cb, seg)])
        plsc.subcore_barrier()
        for h in range(2):
            pltpu.sync_copy(dd_sh.at[h], ddv[h])

        def pass2(kk, _):
            pltpu.sync_copy(src_hbm.at[s].at[kk], srcv)
            pltpu.sync_copy(dst_hbm.at[s].at[kk], dstv)
            for h in range(2):
                for j in range(8):
                    ex16, d16 = edge_ex(h, j)
                    dd16 = plsc.load_gather(ddv[h], [d16])
                    cfr[h, pl.ds(j * 16, 16)] = ex16 * dd16
            for h in range(2):
                pltpu.sync_copy(cfr.at[h],
                                coef_hbm.at[s].at[kk].at[2 * c + h])
            return 0

        lax.fori_loop(0, nblk, pass2, 0)

    return kcoef(a_s2, a_d2, src16, dst16)


def _sc_gat_agg(hg4, src16, dst16, coefg):
    nblk = src16.shape[1]
    rpt = NP // 16

    mesh = plsc.VectorSubcoreMesh(core_axis_name="c", subcore_axis_name="s")

    @functools.partial(
        pl.kernel,
        mesh=mesh,
        compiler_params=pltpu.CompilerParams(needs_layout_passes=False),
        out_type=jax.ShapeDtypeStruct((2, NP, 128), jnp.float32),
        scratch_types=[
            pltpu.VMEM((2, 128), jnp.int32),
            pltpu.VMEM((512,), jnp.float32),
            pltpu.VMEM((64, 128), jnp.float32),
            pltpu.VMEM((64, 128), jnp.float32),
            pltpu.VMEM((128, 128), jnp.float32),
            pltpu.VMEM_SHARED((NP, 128), jnp.float32),
            pltpu.SemaphoreType.DMA,
            pltpu.SemaphoreType.DMA,
            pltpu.SemaphoreType.DMA,
        ],
    )
    def kagg(hg_hbm, src_hbm, dst_hbm, coef_hbm, out_hbm,
             idxv, cfv, bufa, bufb, sumb, acc, sem0, sema, semb):
        c = lax.axis_index("c")
        s = lax.axis_index("s")

        zero = jnp.zeros((16,), jnp.float32)

        def zb(r, _):
            for j in range(8):
                sumb[r, pl.ds(j * 16, 16)] = zero
            return 0

        lax.fori_loop(0, 128, zb, 0)
        rbase = s * rpt
        for t in range(rpt // 128):
            pltpu.sync_copy(sumb, acc.at[pl.ds(rbase + t * 128, 128)])
        plsc.subcore_barrier()

        def scale0(g, _):
            cf16 = cfv[pl.ds(g * 16, 16)]
            for ri in range(16):
                ch = cf16[ri]
                r = g * 16 + ri
                for j in range(8):
                    sl = pl.ds(j * 16, 16)
                    sumb[r, sl] *= ch
            return 0

        def make_accum(h, half, buf):
            def accum(g, _):
                cf16 = cfv[pl.ds(h * 128 + half * 64 + g * 16, 16)]
                for ri in range(16):
                    ch = cf16[ri]
                    r = g * 16 + ri
                    for j in range(8):
                        sl = pl.ds(j * 16, 16)
                        sumb[half * 64 + r, sl] += buf[r, sl] * ch
                return 0
            return accum

        def body(kk, _):
            pltpu.sync_copy(src_hbm.at[s].at[kk], idxv.at[0])
            pltpu.sync_copy(dst_hbm.at[s].at[kk], idxv.at[1])
            pltpu.sync_copy(coef_hbm.at[s].at[kk], cfv)
            sv = idxv.at[0]
            svlo = sv.at[pl.ds(0, 64)]
            svhi = sv.at[pl.ds(64, 64)]
            g0 = pltpu.async_copy(hg_hbm.at[0].at[c].at[sv], sumb, sem0)
            ga = pltpu.async_copy(hg_hbm.at[1].at[c].at[svlo], bufa, sema)
            gb = pltpu.async_copy(hg_hbm.at[1].at[c].at[svhi], bufb, semb)
            g0.wait()
            lax.fori_loop(0, 8, scale0, 0)
            for h in (2, 3, None):
                ga.wait()
                lax.fori_loop(0, 4, make_accum(h - 1 if h else 3, 0, bufa), 0)
                if h is not None:
                    ga = pltpu.async_copy(hg_hbm.at[h].at[c].at[svlo],
                                          bufa, sema)
                gb.wait()
                lax.fori_loop(0, 4, make_accum(h - 1 if h else 3, 1, bufb), 0)
                if h is not None:
                    gb = pltpu.async_copy(hg_hbm.at[h].at[c].at[svhi],
                                          bufb, semb)
            pltpu.sync_copy(sumb, acc.at[idxv.at[1]], add=True)
            return 0

        lax.fori_loop(0, nblk, body, 0)
        plsc.subcore_barrier()
        pltpu.sync_copy(acc.at[pl.ds(rbase, rpt)],
                        out_hbm.at[c].at[pl.ds(rbase, rpt)])

    return kagg(hg4, src16, dst16, coefg.reshape(16, nblk, 512))


def _mm_planes(x, w):
    n, k = x.shape
    br = 1024

    def body(x_ref, w_ref, o_ref):
        o_ref[0, 0, :, :] = jnp.dot(x_ref[...], w_ref[...],
                                    preferred_element_type=jnp.float32)

    return pl.pallas_call(
        body,
        grid=(n // br, 8),
        in_specs=[
            pl.BlockSpec((br, k), lambda i, j: (i, 0)),
            pl.BlockSpec((k, 128), lambda i, j: (0, j)),
        ],
        out_specs=pl.BlockSpec((1, 1, br, 128),
                               lambda i, j: (j // 2, j % 2, i, 0)),
        out_shape=jax.ShapeDtypeStruct((HEADS, 2, n, 128), jnp.float32),
    )(x, w)


def kernel(x, edge_index, batch, W1, b1, g1, be1, W2, b2, g2, be2,
           W3, b3, g3, be3, Wg, att_src, att_dst, bg, Wfc, bfc):
    src = edge_index[0]
    dst = edge_index[1]
    pad_idx = jnp.full((EP - E,), NP - 1, jnp.int32)
    srcp = jnp.concatenate([src, pad_idx])
    dstp = jnp.concatenate([dst, pad_idx])
    src16 = srcp.reshape(16, EP // 16 // EB, EB)
    dst16 = dstp.reshape(16, EP // 16 // EB, EB)
    src32 = srcp.reshape(32, EP // 32 // EB, EB)
    dst32 = dstp.reshape(32, EP // 32 // EB, EB)

    xp = jnp.pad(x, ((0, NP - N), (0, 0)))

    EPD = 32 * 82 * 128
    onesp = jnp.ones((1, NP, 128), jnp.float32)
    dstd = jnp.concatenate([
        dst, batch + 10100,
        jnp.full((EPD - E - N,), NP - 1, jnp.int32)]).reshape(32, 82, 128)
    srcd = jnp.full((EPD,), NP - 1, jnp.int32).reshape(32, 82, 128)
    degp = _sc_gcn_msg(onesp, srcd, dstd, 1)
    degc = degp[0, :, 0] + degp[1, :, 0]
    cnt = degc[10100:10100 + N_GRAPHS]
    dinv = lax.rsqrt(degc[:N] + 1.0)
    dcol = jnp.pad(dinv, (0, NP - N))[:, None]

    def gcn_layer(r, W, b_, g_, be_):
        p = _mm(r, W) * dcol
        m = W.shape[1]
        if m == 128:
            planes = _sc_gcn_msg(p[None], src32, dst32, 1)
            msg = planes[0] + planes[1]
        else:
            pp = p.reshape(NP, 2, 128).transpose(1, 0, 2)
            planes = _sc_gcn_msg(pp, src16, dst16, 2)
            msg = planes.transpose(1, 0, 2).reshape(NP, m)
        y = dcol * (msg + p) + b_
        a, c = _bn_affine(_col_stats(y), g_, be_)
        return jax.nn.relu(y * a + c)

    r1 = gcn_layer(xp, W1, b1, g1, be1)
    r2 = gcn_layer(r1, W2, b2, g2, be2)
    vmask = (jnp.arange(NP) < N).astype(jnp.float32)[:, None]
    r3 = gcn_layer(r2, W3, b3, g3, be3) * vmask

    hg4 = _mm_planes(r3, Wg)
    Wg3 = Wg.reshape(Wg.shape[0], HEADS, GAT_OUT)
    Wasd = jnp.concatenate([(Wg3 * att_src[None]).sum(-1),
                            (Wg3 * att_dst[None]).sum(-1)], axis=1)
    asd = _mm(r3, jnp.pad(Wasd, ((0, 0), (0, 120))))
    a_s = asd[:, :HEADS]
    a_d = asd[:, HEADS:2 * HEADS]
    to2 = lambda t: t.T.reshape(2, 2, NP)

    loops = jnp.arange(NP, dtype=jnp.int32)
    padg = jnp.full((EPG - E - NP,), NP - 1, jnp.int32)
    srcg = jnp.concatenate([src, loops, padg]).reshape(16, EPG // 16 // EB, EB)
    dstg = jnp.concatenate([dst, loops, padg]).reshape(16, EPG // 16 // EB, EB)

    coefg, _dd = _sc_gat_coef(to2(a_s), to2(a_d), srcg, dstg)
    planes_g = _sc_gat_agg(hg4, srcg, dstg, coefg)
    msg_g = planes_g.transpose(1, 0, 2).reshape(NP, GAT_OUT)
    y4 = msg_g * 0.25 + bg
    a4, c4 = _bn_affine(_col_stats(y4), g3, be3)
    r4 = jax.nn.relu(y4 * a4 + c4)

    NPP = 16 * 6 * 128
    srcpool = jnp.concatenate([
        jnp.arange(NP, dtype=jnp.int32),
        jnp.full((NPP - NP,), NP - 1, jnp.int32)]).reshape(16, 6, 128)
    dstpool = jnp.concatenate([
        batch, jnp.full((NPP - N,), NP - 1, jnp.int32)]).reshape(16, 6, 128)
    r4planes = r4.reshape(NP, 2, 128).transpose(1, 0, 2)
    poolp = _sc_gcn_msg(r4planes, srcpool, dstpool, 2)
    sums = poolp.transpose(1, 0, 2).reshape(NP, GAT_OUT)[:N_GRAPHS]
    pooled = sums / jnp.clip(cnt, 1.0)[:, None]

    def fc_body(p_ref, w_ref, b_ref, o_ref):
        o_ref[...] = jax.nn.relu(
            jnp.dot(p_ref[...], w_ref[...],
                    preferred_element_type=jnp.float32) + b_ref[...])

    out = pl.pallas_call(
        fc_body,
        out_shape=jax.ShapeDtypeStruct((N_GRAPHS, Wfc.shape[1]), jnp.float32),
    )(pooled, Wfc, bfc[None, :])
    return out


# speedup vs baseline: 3.0373x; 3.0373x over previous
import functools

import jax
import jax.numpy as jnp
from jax import lax
from jax.experimental import pallas as pl
from jax.experimental.pallas import tpu as pltpu
from jax.experimental.pallas import tpu_sc as plsc

N = 10000
E = 320000
HEADS = 4
GAT_OUT = 256
N_GRAPHS = 64
BR = 1000

NP = 10240
EP = 327680
EPG = 331776
EB = 128


def _sc_gcn_msg(p_planes, src3, dst3, n_planes):
    n_chunks = src3.shape[0]
    nblk = src3.shape[1]
    rpt = NP // 16

    mesh = plsc.VectorSubcoreMesh(core_axis_name="c", subcore_axis_name="s")

    @functools.partial(
        pl.kernel,
        mesh=mesh,
        compiler_params=pltpu.CompilerParams(needs_layout_passes=False),
        out_type=jax.ShapeDtypeStruct((2, NP, 128), jnp.float32),
        scratch_types=[
            pltpu.VMEM((2, EB), jnp.int32),
            pltpu.VMEM((2, EB), jnp.int32),
            pltpu.VMEM((EB, 128), jnp.float32),
            pltpu.VMEM((EB, 128), jnp.float32),
            pltpu.VMEM_SHARED((NP, 128), jnp.float32),
            pltpu.SemaphoreType.DMA,
            pltpu.SemaphoreType.DMA,
        ],
    )
    def kmsg(p_hbm, src_hbm, dst_hbm, out_hbm, idxa, idxb, rowa, rowb,
             acc, sema, semb):
        c = lax.axis_index("c")
        s = lax.axis_index("s")
        chunk = s if n_chunks == 16 else c * 16 + s
        plane = c if n_planes == 2 else 0

        zero = jnp.zeros((16,), jnp.float32)

        def zbody(r, _):
            for j in range(8):
                rowa[r, pl.ds(j * 16, 16)] = zero
            return 0

        lax.fori_loop(0, EB, zbody, 0)
        rbase = s * rpt
        for t in range(rpt // EB):
            pltpu.sync_copy(rowa, acc.at[pl.ds(rbase + t * EB, EB)])
        plsc.subcore_barrier()

        idx = [idxa, idxb]
        row = [rowa, rowb]
        sem = [sema, semb]

        def load_and_fire(blk, b):
            pltpu.sync_copy(src_hbm.at[chunk].at[blk], idx[b].at[0])
            pltpu.sync_copy(dst_hbm.at[chunk].at[blk], idx[b].at[1])
            pltpu.async_copy(p_hbm.at[plane].at[idx[b].at[0]],
                             row[b], sem[b])

        def drain_and_scatter(b):
            pltpu.make_async_copy(p_hbm.at[plane].at[idx[b].at[0]],
                                  row[b], sem[b]).wait()
            pltpu.sync_copy(row[b], acc.at[idx[b].at[1]], add=True)

        load_and_fire(0, 0)

        def body(kk, _):
            blk2 = kk * 2
            load_and_fire(blk2 + 1, 1)
            drain_and_scatter(0)
            nxt = jnp.minimum(blk2 + 2, nblk - 1)
            load_and_fire(nxt, 0)
            drain_and_scatter(1)
            return 0

        lax.fori_loop(0, nblk // 2, body, 0)
        pltpu.make_async_copy(p_hbm.at[plane].at[idxa.at[0]],
                              rowa, sema).wait()
        plsc.subcore_barrier()
        pltpu.sync_copy(acc.at[pl.ds(rbase, rpt)],
                        out_hbm.at[c].at[pl.ds(rbase, rpt)])

    return kmsg(p_planes, src3, dst3)


def _mm(x, w):
    n, k = x.shape
    m = w.shape[1]
    br = 1024 if n % 1024 == 0 else BR

    def body(x_ref, w_ref, o_ref):
        o_ref[...] = jnp.dot(x_ref[...], w_ref[...],
                             preferred_element_type=jnp.float32)

    return pl.pallas_call(
        body,
        grid=(n // br,),
        in_specs=[
            pl.BlockSpec((br, k), lambda i: (i, 0)),
            pl.BlockSpec((k, m), lambda i: (0, 0)),
        ],
        out_specs=pl.BlockSpec((br, m), lambda i: (i, 0)),
        out_shape=jax.ShapeDtypeStruct((n, m), jnp.float32),
    )(x, w)


def _col_stats(y):
    n, m = y.shape

    def body(y_ref, s_ref):
        @pl.when(pl.program_id(0) == 0)
        def _():
            s_ref[...] = jnp.zeros_like(s_ref)

        yb = y_ref[...]
        s_ref[0:1, :] += jnp.sum(yb, 0, keepdims=True)
        s_ref[1:2, :] += jnp.sum(yb * yb, 0, keepdims=True)

    return pl.pallas_call(
        body,
        grid=(n // BR,),
        in_specs=[pl.BlockSpec((BR, m), lambda i: (i, 0))],
        out_specs=pl.BlockSpec((8, m), lambda i: (0, 0)),
        out_shape=jax.ShapeDtypeStruct((8, m), jnp.float32),
    )(y)


def _bn_affine(stats, g, be):
    s1 = stats[0]
    s2 = stats[1]
    mean = s1 / N
    var = s2 / N - mean * mean
    a = g * lax.rsqrt(var + 1e-5)
    c = be - mean * a
    return a, c


def _sc_gat_coef(a_s2, a_d2, src16, dst16):
    nblk = src16.shape[1]
    seg = NP // 16

    mesh = plsc.VectorSubcoreMesh(core_axis_name="c", subcore_axis_name="s")

    @functools.partial(
        pl.kernel,
        mesh=mesh,
        compiler_params=pltpu.CompilerParams(needs_layout_passes=False),
        out_type=(
            jax.ShapeDtypeStruct((16, nblk, HEADS, 128), jnp.float32),
            jax.ShapeDtypeStruct((HEADS, NP), jnp.float32),
        ),
        scratch_types=[
            pltpu.VMEM((NP,), jnp.float32),
            pltpu.VMEM((NP,), jnp.float32),
            pltpu.VMEM((NP,), jnp.float32),
            pltpu.VMEM((NP,), jnp.float32),
            pltpu.VMEM((NP,), jnp.float32),
            pltpu.VMEM((NP,), jnp.float32),
            pltpu.VMEM((NP,), jnp.float32),
            pltpu.VMEM((NP,), jnp.float32),
            pltpu.VMEM((128,), jnp.int32),
            pltpu.VMEM((128,), jnp.int32),
            pltpu.VMEM((2, 128), jnp.float32),
            pltpu.VMEM((seg,), jnp.float32),
            pltpu.VMEM((seg,), jnp.float32),
            pltpu.VMEM((seg,), jnp.float32),
            pltpu.VMEM_SHARED((16, 2, NP), jnp.float32),
            pltpu.VMEM_SHARED((2, NP), jnp.float32),
        ],
    )
    def kcoef(as_hbm, ad_hbm, src_hbm, dst_hbm, coef_hbm, dd_hbm,
          asv0, asv1, adv0, adv1, denl0, denl1, ddv0, ddv1,
          srcv, dstv, cfr, sum0, sum1, tmpv, den_sh, dd_sh):
        c = lax.axis_index("c")
        s = lax.axis_index("s")
        asv = [asv0, asv1]
        adv = [adv0, adv1]
        denl = [denl0, denl1]
        ddv = [ddv0, ddv1]
        sumv = [sum0, sum1]
        for h in range(2):
            pltpu.sync_copy(as_hbm.at[c].at[h], asv[h])
            pltpu.sync_copy(ad_hbm.at[c].at[h], adv[h])

        zero = jnp.zeros((16,), jnp.float32)

        def zb(i, _):
            denl0[pl.ds(i * 16, 16)] = zero
            denl1[pl.ds(i * 16, 16)] = zero
            return 0

        lax.fori_loop(0, NP // 16, zb, 0)

        def edge_ex(h, j):
            s16 = srcv[pl.ds(j * 16, 16)]
            d16 = dstv[pl.ds(j * 16, 16)]
            a16 = plsc.load_gather(asv[h], [s16])
            b16 = plsc.load_gather(adv[h], [d16])
            e16 = a16 + b16
            e16 = jnp.maximum(e16, 0.2 * e16)
            return jnp.exp(e16), d16

        def pass1(kk, _):
            pltpu.sync_copy(src_hbm.at[s].at[kk], srcv)
            pltpu.sync_copy(dst_hbm.at[s].at[kk], dstv)
            for h in range(2):
                for j in range(8):
                    ex16, d16 = edge_ex(h, j)
                    plsc.addupdate_scatter(denl[h], [d16], ex16)
            return 0

        lax.fori_loop(0, nblk, pass1, 0)
        for h in range(2):
            pltpu.sync_copy(denl[h], den_sh.at[s].at[h])
        plsc.subcore_barrier()

        cb = s * seg
        for h in range(2):
            sv = sumv[h]
            pltpu.sync_copy(den_sh.at[0].at[h].at[pl.ds(cb, seg)], tmpv)

            def cpy(i, _):
                sv[pl.ds(i * 16, 16)] = tmpv[pl.ds(i * 16, 16)]
                return 0

            lax.fori_loop(0, seg // 16, cpy, 0)

            def red_t(t, _):
                pltpu.sync_copy(den_sh.at[t].at[h].at[pl.ds(cb, seg)], tmpv)

                def add_i(i, _):
                    sl = pl.ds(i * 16, 16)
                    sv[sl] += tmpv[sl]
                    return 0

                lax.fori_loop(0, seg // 16, add_i, 0)
                return 0

            lax.fori_loop(1, 16, red_t, 0)

            def inv_i(i, _):
                sl = pl.ds(i * 16, 16)
                sv[sl] = 1.0 / (sv[sl] + 1e-16)
                return 0

            lax.fori_loop(0, seg // 16, inv_i, 0)
            pltpu.sync_copy(sv, dd_sh.at[h].at[pl.ds(cb, seg)])
            pltpu.sync_copy(sv, dd_hbm.at[2 * c + h].at[pl.ds(cb, seg)])
        plsc.subcore_barrier()
        for h in range(2):
            pltpu.sync_copy(dd_sh.at[h], ddv[h])

        def pass2(kk, _):
            pltpu.sync_copy(src_hbm.at[s].at[kk], srcv)
            pltpu.sync_copy(dst_hbm.at[s].at[kk], dstv)
            for h in range(2):
                for j in range(8):
                    ex16, d16 = edge_ex(h, j)
                    dd16 = plsc.load_gather(ddv[h], [d16])
                    cfr[h, pl.ds(j * 16, 16)] = ex16 * dd16
            for h in range(2):
                pltpu.sync_copy(cfr.at[h],
                                coef_hbm.at[s].at[kk].at[2 * c + h])
            return 0

        lax.fori_loop(0, nblk, pass2, 0)

    return kcoef(a_s2, a_d2, src16, dst16)


def _sc_gat_agg(hg4, src16, dst16, coefg):
    nblk = src16.shape[1]
    rpt = NP // 16

    mesh = plsc.VectorSubcoreMesh(core_axis_name="c", subcore_axis_name="s")

    @functools.partial(
        pl.kernel,
        mesh=mesh,
        compiler_params=pltpu.CompilerParams(needs_layout_passes=False),
        out_type=jax.ShapeDtypeStruct((2, NP, 128), jnp.float32),
        scratch_types=[
            pltpu.VMEM((2, 128), jnp.int32),
            pltpu.VMEM((512,), jnp.float32),
            pltpu.VMEM((64, 128), jnp.float32),
            pltpu.VMEM((64, 128), jnp.float32),
            pltpu.VMEM((128, 128), jnp.float32),
            pltpu.VMEM_SHARED((NP, 128), jnp.float32),
            pltpu.SemaphoreType.DMA,
            pltpu.SemaphoreType.DMA,
            pltpu.SemaphoreType.DMA,
        ],
    )
    def kagg(hg_hbm, src_hbm, dst_hbm, coef_hbm, out_hbm,
             idxv, cfv, bufa, bufb, sumb, acc, sem0, sema, semb):
        c = lax.axis_index("c")
        s = lax.axis_index("s")

        zero = jnp.zeros((16,), jnp.float32)

        def zb(r, _):
            for j in range(8):
                sumb[r, pl.ds(j * 16, 16)] = zero
            return 0

        lax.fori_loop(0, 128, zb, 0)
        rbase = s * rpt
        for t in range(rpt // 128):
            pltpu.sync_copy(sumb, acc.at[pl.ds(rbase + t * 128, 128)])
        plsc.subcore_barrier()

        def scale0(g, _):
            cf16 = cfv[pl.ds(g * 16, 16)]
            for ri in range(16):
                ch = cf16[ri]
                r = g * 16 + ri
                for j in range(8):
                    sl = pl.ds(j * 16, 16)
                    sumb[r, sl] *= ch
            return 0

        def make_accum(h, half, buf):
            def accum(g, _):
                cf16 = cfv[pl.ds(h * 128 + half * 64 + g * 16, 16)]
                for ri in range(16):
                    ch = cf16[ri]
                    r = g * 16 + ri
                    for j in range(8):
                        sl = pl.ds(j * 16, 16)
                        sumb[half * 64 + r, sl] += buf[r, sl] * ch
                return 0
            return accum

        def body(kk, _):
            pltpu.sync_copy(src_hbm.at[s].at[kk], idxv.at[0])
            pltpu.sync_copy(dst_hbm.at[s].at[kk], idxv.at[1])
            pltpu.sync_copy(coef_hbm.at[s].at[kk], cfv)
            sv = idxv.at[0]
            svlo = sv.at[pl.ds(0, 64)]
            svhi = sv.at[pl.ds(64, 64)]
            g0 = pltpu.async_copy(hg_hbm.at[0].at[c].at[sv], sumb, sem0)
            ga = pltpu.async_copy(hg_hbm.at[1].at[c].at[svlo], bufa, sema)
            gb = pltpu.async_copy(hg_hbm.at[1].at[c].at[svhi], bufb, semb)
            g0.wait()
            lax.fori_loop(0, 8, scale0, 0)
            for h in (2, 3, None):
                ga.wait()
                lax.fori_loop(0, 4, make_accum(h - 1 if h else 3, 0, bufa), 0)
                if h is not None:
                    ga = pltpu.async_copy(hg_hbm.at[h].at[c].at[svlo],
                                          bufa, sema)
                gb.wait()
                lax.fori_loop(0, 4, make_accum(h - 1 if h else 3, 1, bufb), 0)
                if h is not None:
                    gb = pltpu.async_copy(hg_hbm.at[h].at[c].at[svhi],
                                          bufb, semb)
            pltpu.sync_copy(sumb, acc.at[idxv.at[1]], add=True)
            return 0

        lax.fori_loop(0, nblk, body, 0)
        plsc.subcore_barrier()
        pltpu.sync_copy(acc.at[pl.ds(rbase, rpt)],
                        out_hbm.at[c].at[pl.ds(rbase, rpt)])

    return kagg(hg4, src16, dst16, coefg.reshape(16, nblk, 512))


def _mm_planes(x, w):
    n, k = x.shape
    br = 1024

    def body(x_ref, w_ref, o_ref):
        o_ref[0, 0, :, :] = jnp.dot(x_ref[...], w_ref[...],
                                    preferred_element_type=jnp.float32)

    return pl.pallas_call(
        body,
        grid=(n // br, 8),
        in_specs=[
            pl.BlockSpec((br, k), lambda i, j: (i, 0)),
            pl.BlockSpec((k, 128), lambda i, j: (0, j)),
        ],
        out_specs=pl.BlockSpec((1, 1, br, 128),
                               lambda i, j: (j // 2, j % 2, i, 0)),
        out_shape=jax.ShapeDtypeStruct((HEADS, 2, n, 128), jnp.float32),
    )(x, w)


def kernel(x, edge_index, batch, W1, b1, g1, be1, W2, b2, g2, be2,
           W3, b3, g3, be3, Wg, att_src, att_dst, bg, Wfc, bfc):
    src = edge_index[0]
    dst = edge_index[1]
    pad_idx = jnp.full((EP - E,), NP - 1, jnp.int32)
    srcp = jnp.concatenate([src, pad_idx])
    dstp = jnp.concatenate([dst, pad_idx])
    src16 = srcp.reshape(16, EP // 16 // EB, EB)
    dst16 = dstp.reshape(16, EP // 16 // EB, EB)
    src32 = srcp.reshape(32, EP // 32 // EB, EB)
    dst32 = dstp.reshape(32, EP // 32 // EB, EB)

    xp = jnp.pad(x, ((0, NP - N), (0, 0)))

    deg = jnp.zeros((N,), jnp.float32).at[dst].add(1.0) + 1.0
    dinv = lax.rsqrt(deg)
    dcol = jnp.pad(dinv, (0, NP - N))[:, None]

    def gcn_layer(r, W, b_, g_, be_):
        p = _mm(r, W) * dcol
        m = W.shape[1]
        if m == 128:
            planes = _sc_gcn_msg(p[None], src32, dst32, 1)
            msg = planes[0] + planes[1]
        else:
            pp = p.reshape(NP, 2, 128).transpose(1, 0, 2)
            planes = _sc_gcn_msg(pp, src16, dst16, 2)
            msg = planes.transpose(1, 0, 2).reshape(NP, m)
        y = dcol * (msg + p) + b_
        a, c = _bn_affine(_col_stats(y), g_, be_)
        return jax.nn.relu(y * a + c)

    r1 = gcn_layer(xp, W1, b1, g1, be1)
    r2 = gcn_layer(r1, W2, b2, g2, be2)
    vmask = (jnp.arange(NP) < N).astype(jnp.float32)[:, None]
    r3 = gcn_layer(r2, W3, b3, g3, be3) * vmask

    hg4 = _mm_planes(r3, Wg)
    Wg3 = Wg.reshape(Wg.shape[0], HEADS, GAT_OUT)
    Wasd = jnp.concatenate([(Wg3 * att_src[None]).sum(-1),
                            (Wg3 * att_dst[None]).sum(-1)], axis=1)
    asd = _mm(r3, jnp.pad(Wasd, ((0, 0), (0, 120))))
    a_s = asd[:, :HEADS]
    a_d = asd[:, HEADS:2 * HEADS]
    to2 = lambda t: t.T.reshape(2, 2, NP)

    loops = jnp.arange(NP, dtype=jnp.int32)
    padg = jnp.full((EPG - E - NP,), NP - 1, jnp.int32)
    srcg = jnp.concatenate([src, loops, padg]).reshape(16, EPG // 16 // EB, EB)
    dstg = jnp.concatenate([dst, loops, padg]).reshape(16, EPG // 16 // EB, EB)

    coefg, _dd = _sc_gat_coef(to2(a_s), to2(a_d), srcg, dstg)
    planes_g = _sc_gat_agg(hg4, srcg, dstg, coefg)
    msg_g = planes_g.transpose(1, 0, 2).reshape(NP, GAT_OUT)
    y4 = msg_g * 0.25 + bg
    a4, c4 = _bn_affine(_col_stats(y4), g3, be3)
    r4 = jax.nn.relu(y4 * a4 + c4)[:N]

    sums = jax.ops.segment_sum(r4, batch, num_segments=N_GRAPHS)
    cnt = jax.ops.segment_sum(jnp.ones((N,), jnp.float32), batch,
                              num_segments=N_GRAPHS)
    pooled = sums / jnp.clip(cnt, 1.0)[:, None]

    def fc_body(p_ref, w_ref, b_ref, o_ref):
        o_ref[...] = jax.nn.relu(
            jnp.dot(p_ref[...], w_ref[...],
                    preferred_element_type=jnp.float32) + b_ref[...])

    out = pl.pallas_call(
        fc_body,
        out_shape=jax.ShapeDtypeStruct((N_GRAPHS, Wfc.shape[1]), jnp.float32),
    )(pooled, Wfc, bfc[None, :])
    return out
